# Initial kernel scaffold; baseline (speedup 1.0000x reference)
import jax
import jax.numpy as jnp
from jax.experimental import pallas as pl


def kernel(h, W1, b1, Wb, bb, Wc, bc, Wcls, bcls):
    raise NotImplementedError("write your pallas kernel here")


# speedup vs baseline: 1.2540x; 1.2540x over previous
import functools

import jax
import jax.numpy as jnp
from jax.experimental import pallas as pl
from jax.experimental.pallas import tpu as pltpu


def _mil_body(nb, bn, h_ref, w1t_ref, b1_ref, wbt_ref, bb_ref, wc_ref,
              bc_ref, wclst_ref, bcls_ref,
              araw_ref, logits_ref, yprob_ref, yhat_ref,
              bestv_ref, bestf_ref):
    i = pl.program_id(0)

    @pl.when(i == 0)
    def _init():
        bestv_ref[0, 0] = -jnp.inf

    x = jnp.dot(h_ref[...], w1t_ref[...], preferred_element_type=jnp.float32)
    x = jnp.maximum(x + b1_ref[...], 0.0)
    gate = jnp.dot(x, wbt_ref[...], preferred_element_type=jnp.float32)
    gate = jax.nn.sigmoid(gate + bb_ref[...])
    feat = x * gate
    a = jnp.sum(feat * wc_ref[...], axis=1, keepdims=True)
    a = a + bc_ref[0, 0]
    araw_ref[...] = a

    av = a[:, 0]
    bmax = jnp.max(av)
    bidx = jnp.argmax(av)

    @pl.when(bmax > bestv_ref[0, 0])
    def _update():
        bestv_ref[0, 0] = bmax
        rows = jax.lax.broadcasted_iota(jnp.int32, feat.shape, 0)
        bestf_ref[...] = jnp.sum(
            jnp.where(rows == bidx, feat, 0.0), axis=0, keepdims=True)

    @pl.when(i == nb - 1)
    def _finish():
        m = bestf_ref[...]
        logits = jnp.dot(m, wclst_ref[...],
                         preferred_element_type=jnp.float32) + bcls_ref[...]
        logits_ref[...] = logits
        yprob_ref[...] = jax.nn.softmax(logits, axis=1)
        yhat_ref[...] = jnp.argmax(logits, axis=1).reshape(1, 1).astype(jnp.int32)


@jax.jit
def kernel(h, W1, b1, Wb, bb, Wc, bc, Wcls, bcls):
    N, E = h.shape
    H = W1.shape[0]
    n_classes = Wcls.shape[0]

    bn = 1000 if N % 1000 == 0 else None
    if bn is None:
        for cand in (500, 400, 250, 200, 125, 100, 50, 25, 8, 1):
            if N % cand == 0:
                bn = cand
                break
    nb = N // bn

    w1t = W1.T
    wbt = Wb.T
    wclst = Wcls.T
    b1r = b1.reshape(1, H)
    bbr = bb.reshape(1, H)
    wcr = Wc.reshape(1, H)
    bcr = bc.reshape(1, 1)
    bclsr = bcls.reshape(1, n_classes)

    araw, logits, yprob, yhat = pl.pallas_call(
        functools.partial(_mil_body, nb, bn),
        grid=(nb,),
        in_specs=[
            pl.BlockSpec((bn, E), lambda i: (i, 0)),
            pl.BlockSpec((E, H), lambda i: (0, 0)),
            pl.BlockSpec((1, H), lambda i: (0, 0)),
            pl.BlockSpec((H, H), lambda i: (0, 0)),
            pl.BlockSpec((1, H), lambda i: (0, 0)),
            pl.BlockSpec((1, H), lambda i: (0, 0)),
            pl.BlockSpec((1, 1), lambda i: (0, 0)),
            pl.BlockSpec((H, n_classes), lambda i: (0, 0)),
            pl.BlockSpec((1, n_classes), lambda i: (0, 0)),
        ],
        out_specs=[
            pl.BlockSpec((bn, 1), lambda i: (i, 0)),
            pl.BlockSpec((1, n_classes), lambda i: (0, 0)),
            pl.BlockSpec((1, n_classes), lambda i: (0, 0)),
            pl.BlockSpec((1, 1), lambda i: (0, 0)),
        ],
        out_shape=[
            jax.ShapeDtypeStruct((N, 1), jnp.float32),
            jax.ShapeDtypeStruct((1, n_classes), jnp.float32),
            jax.ShapeDtypeStruct((1, n_classes), jnp.float32),
            jax.ShapeDtypeStruct((1, 1), jnp.int32),
        ],
        scratch_shapes=[
            pltpu.SMEM((1, 1), jnp.float32),
            pltpu.VMEM((1, H), jnp.float32),
        ],
        compiler_params=pltpu.CompilerParams(
            dimension_semantics=("arbitrary",),
        ),
    )(h, w1t, b1r, wbt, bbr, wcr, bcr, wclst, bclsr)

    return logits, yprob, yhat, araw.reshape(1, N)


# speedup vs baseline: 1.2580x; 1.0032x over previous
import functools

import jax
import jax.numpy as jnp
from jax.experimental import pallas as pl
from jax.experimental.pallas import tpu as pltpu


def _mil_body(nb, bn, h_ref, w1t_ref, b1_ref, wbt_ref, bb_ref, wc_ref,
              bc_ref, wclst_ref, bcls_ref,
              araw_ref, logits_ref, yprob_ref, yhat_ref,
              bestv_ref, bestf_ref):
    i = pl.program_id(0)

    @pl.when(i == 0)
    def _init():
        bestv_ref[0, 0] = -jnp.inf

    x = jnp.dot(h_ref[...].astype(jnp.bfloat16), w1t_ref[...],
                preferred_element_type=jnp.float32)
    x = jnp.maximum(x + b1_ref[...], 0.0)
    gate = jnp.dot(x.astype(jnp.bfloat16), wbt_ref[...],
                   preferred_element_type=jnp.float32)
    gate = jax.nn.sigmoid(gate + bb_ref[...])
    feat = x * gate
    a = jnp.sum(feat * wc_ref[...], axis=1, keepdims=True)
    a = a + bc_ref[0, 0]
    araw_ref[...] = a

    av = a[:, 0]
    bmax = jnp.max(av)
    bidx = jnp.argmax(av)

    @pl.when(bmax > bestv_ref[0, 0])
    def _update():
        bestv_ref[0, 0] = bmax
        rows = jax.lax.broadcasted_iota(jnp.int32, feat.shape, 0)
        bestf_ref[...] = jnp.sum(
            jnp.where(rows == bidx, feat, 0.0), axis=0, keepdims=True)

    @pl.when(i == nb - 1)
    def _finish():
        m = bestf_ref[...]
        logits = jnp.dot(m, wclst_ref[...],
                         preferred_element_type=jnp.float32) + bcls_ref[...]
        logits_ref[...] = logits
        yprob_ref[...] = jax.nn.softmax(logits, axis=1)
        yhat_ref[...] = jnp.argmax(logits, axis=1).reshape(1, 1).astype(jnp.int32)


@jax.jit
def kernel(h, W1, b1, Wb, bb, Wc, bc, Wcls, bcls):
    N, E = h.shape
    H = W1.shape[0]
    n_classes = Wcls.shape[0]

    bn = 1000 if N % 1000 == 0 else None
    if bn is None:
        for cand in (500, 400, 250, 200, 125, 100, 50, 25, 8, 1):
            if N % cand == 0:
                bn = cand
                break
    nb = N // bn

    w1t = W1.T.astype(jnp.bfloat16)
    wbt = Wb.T.astype(jnp.bfloat16)
    wclst = Wcls.T
    b1r = b1.reshape(1, H)
    bbr = bb.reshape(1, H)
    wcr = Wc.reshape(1, H)
    bcr = bc.reshape(1, 1)
    bclsr = bcls.reshape(1, n_classes)

    araw, logits, yprob, yhat = pl.pallas_call(
        functools.partial(_mil_body, nb, bn),
        grid=(nb,),
        in_specs=[
            pl.BlockSpec((bn, E), lambda i: (i, 0)),
            pl.BlockSpec((E, H), lambda i: (0, 0)),
            pl.BlockSpec((1, H), lambda i: (0, 0)),
            pl.BlockSpec((H, H), lambda i: (0, 0)),
            pl.BlockSpec((1, H), lambda i: (0, 0)),
            pl.BlockSpec((1, H), lambda i: (0, 0)),
            pl.BlockSpec((1, 1), lambda i: (0, 0)),
            pl.BlockSpec((H, n_classes), lambda i: (0, 0)),
            pl.BlockSpec((1, n_classes), lambda i: (0, 0)),
        ],
        out_specs=[
            pl.BlockSpec((bn, 1), lambda i: (i, 0)),
            pl.BlockSpec((1, n_classes), lambda i: (0, 0)),
            pl.BlockSpec((1, n_classes), lambda i: (0, 0)),
            pl.BlockSpec((1, 1), lambda i: (0, 0)),
        ],
        out_shape=[
            jax.ShapeDtypeStruct((N, 1), jnp.float32),
            jax.ShapeDtypeStruct((1, n_classes), jnp.float32),
            jax.ShapeDtypeStruct((1, n_classes), jnp.float32),
            jax.ShapeDtypeStruct((1, 1), jnp.int32),
        ],
        scratch_shapes=[
            pltpu.SMEM((1, 1), jnp.float32),
            pltpu.VMEM((1, H), jnp.float32),
        ],
        compiler_params=pltpu.CompilerParams(
            dimension_semantics=("arbitrary",),
        ),
    )(h, w1t, b1r, wbt, bbr, wcr, bcr, wclst, bclsr)

    return logits, yprob, yhat, araw.reshape(1, N)


# speedup vs baseline: 1.4936x; 1.1873x over previous
import functools

import jax
import jax.numpy as jnp
from jax.experimental import pallas as pl
from jax.experimental.pallas import tpu as pltpu


def _mil_body(nb, bn, h_ref, w1t_ref, b1_ref, wbt_ref, bb_ref, wc_ref,
              bc_ref, wclst_ref, bcls_ref,
              araw_ref, logits_ref, yprob_ref, yhat_ref,
              bestv_ref, bestf_ref):
    i = pl.program_id(0)

    @pl.when(i == 0)
    def _init():
        bestv_ref[0, 0] = -jnp.inf

    x = jnp.dot(h_ref[...].astype(jnp.bfloat16), w1t_ref[...],
                preferred_element_type=jnp.float32)
    x = jnp.maximum(x + b1_ref[...], 0.0)
    gate = jnp.dot(x.astype(jnp.bfloat16), wbt_ref[...],
                   preferred_element_type=jnp.float32)
    gate = jax.nn.sigmoid(gate + bb_ref[...])
    feat = x * gate
    a = jnp.sum(feat * wc_ref[...], axis=1, keepdims=True)
    a = a + bc_ref[0, 0]
    araw_ref[...] = a

    av = a[:, 0]
    bmax = jnp.max(av)
    bidx = jnp.argmax(av)

    @pl.when(bmax > bestv_ref[0, 0])
    def _update():
        bestv_ref[0, 0] = bmax
        rows = jax.lax.broadcasted_iota(jnp.int32, feat.shape, 0)
        bestf_ref[...] = jnp.sum(
            jnp.where(rows == bidx, feat, 0.0), axis=0, keepdims=True)

    @pl.when(i == nb - 1)
    def _finish():
        m = bestf_ref[...]
        logits = jnp.dot(m, wclst_ref[...],
                         preferred_element_type=jnp.float32) + bcls_ref[...]
        logits_ref[...] = logits
        yprob_ref[...] = jax.nn.softmax(logits, axis=1)
        yhat_ref[...] = jnp.argmax(logits, axis=1).reshape(1, 1).astype(jnp.int32)


@jax.jit
def kernel(h, W1, b1, Wb, bb, Wc, bc, Wcls, bcls):
    N, E = h.shape
    H = W1.shape[0]
    n_classes = Wcls.shape[0]

    bn = 2000 if N % 2000 == 0 else None
    if bn is None:
        for cand in (500, 400, 250, 200, 125, 100, 50, 25, 8, 1):
            if N % cand == 0:
                bn = cand
                break
    nb = N // bn

    w1t = W1.T.astype(jnp.bfloat16)
    wbt = Wb.T.astype(jnp.bfloat16)
    wclst = Wcls.T
    b1r = b1.reshape(1, H)
    bbr = bb.reshape(1, H)
    wcr = Wc.reshape(1, H)
    bcr = bc.reshape(1, 1)
    bclsr = bcls.reshape(1, n_classes)

    araw, logits, yprob, yhat = pl.pallas_call(
        functools.partial(_mil_body, nb, bn),
        grid=(nb,),
        in_specs=[
            pl.BlockSpec((bn, E), lambda i: (i, 0)),
            pl.BlockSpec((E, H), lambda i: (0, 0)),
            pl.BlockSpec((1, H), lambda i: (0, 0)),
            pl.BlockSpec((H, H), lambda i: (0, 0)),
            pl.BlockSpec((1, H), lambda i: (0, 0)),
            pl.BlockSpec((1, H), lambda i: (0, 0)),
            pl.BlockSpec((1, 1), lambda i: (0, 0)),
            pl.BlockSpec((H, n_classes), lambda i: (0, 0)),
            pl.BlockSpec((1, n_classes), lambda i: (0, 0)),
        ],
        out_specs=[
            pl.BlockSpec((bn, 1), lambda i: (i, 0)),
            pl.BlockSpec((1, n_classes), lambda i: (0, 0)),
            pl.BlockSpec((1, n_classes), lambda i: (0, 0)),
            pl.BlockSpec((1, 1), lambda i: (0, 0)),
        ],
        out_shape=[
            jax.ShapeDtypeStruct((N, 1), jnp.float32),
            jax.ShapeDtypeStruct((1, n_classes), jnp.float32),
            jax.ShapeDtypeStruct((1, n_classes), jnp.float32),
            jax.ShapeDtypeStruct((1, 1), jnp.int32),
        ],
        scratch_shapes=[
            pltpu.SMEM((1, 1), jnp.float32),
            pltpu.VMEM((1, H), jnp.float32),
        ],
        compiler_params=pltpu.CompilerParams(
            dimension_semantics=("arbitrary",),
        ),
    )(h, w1t, b1r, wbt, bbr, wcr, bcr, wclst, bclsr)

    return logits, yprob, yhat, araw.reshape(1, N)


# speedup vs baseline: 1.6282x; 1.0901x over previous
import functools

import jax
import jax.numpy as jnp
from jax.experimental import pallas as pl
from jax.experimental.pallas import tpu as pltpu


def _mil_body(nb, bn, h_ref, w1t_ref, b1_ref, wbt_ref, bb_ref, wc_ref,
              bc_ref, wclst_ref, bcls_ref,
              araw_ref, logits_ref, yprob_ref, yhat_ref,
              bestv_ref, bestf_ref):
    i = pl.program_id(0)

    @pl.when(i == 0)
    def _init():
        bestv_ref[0, 0] = -jnp.inf

    x = jnp.dot(h_ref[...].astype(jnp.bfloat16), w1t_ref[...],
                preferred_element_type=jnp.float32)
    x = jnp.maximum(x + b1_ref[...], 0.0)
    gate = jnp.dot(x.astype(jnp.bfloat16), wbt_ref[...],
                   preferred_element_type=jnp.float32)
    gate = jax.nn.sigmoid(gate + bb_ref[...])
    feat = x * gate
    a = jnp.sum(feat * wc_ref[...], axis=1, keepdims=True)
    a = a + bc_ref[0, 0]
    araw_ref[...] = a

    av = a[:, 0]
    bmax = jnp.max(av)
    bidx = jnp.argmax(av)

    @pl.when(bmax > bestv_ref[0, 0])
    def _update():
        bestv_ref[0, 0] = bmax
        rows = jax.lax.broadcasted_iota(jnp.int32, feat.shape, 0)
        bestf_ref[...] = jnp.sum(
            jnp.where(rows == bidx, feat, 0.0), axis=0, keepdims=True)

    @pl.when(i == nb - 1)
    def _finish():
        m = bestf_ref[...]
        logits = jnp.dot(m, wclst_ref[...],
                         preferred_element_type=jnp.float32) + bcls_ref[...]
        logits_ref[...] = logits
        yprob_ref[...] = jax.nn.softmax(logits, axis=1)
        yhat_ref[...] = jnp.argmax(logits, axis=1).reshape(1, 1).astype(jnp.int32)


@jax.jit
def kernel(h, W1, b1, Wb, bb, Wc, bc, Wcls, bcls):
    N, E = h.shape
    H = W1.shape[0]
    n_classes = Wcls.shape[0]

    bn = 5000 if N % 5000 == 0 else None
    if bn is None:
        for cand in (500, 400, 250, 200, 125, 100, 50, 25, 8, 1):
            if N % cand == 0:
                bn = cand
                break
    nb = N // bn

    w1t = W1.T.astype(jnp.bfloat16)
    wbt = Wb.T.astype(jnp.bfloat16)
    wclst = Wcls.T
    b1r = b1.reshape(1, H)
    bbr = bb.reshape(1, H)
    wcr = Wc.reshape(1, H)
    bcr = bc.reshape(1, 1)
    bclsr = bcls.reshape(1, n_classes)

    araw, logits, yprob, yhat = pl.pallas_call(
        functools.partial(_mil_body, nb, bn),
        grid=(nb,),
        in_specs=[
            pl.BlockSpec((bn, E), lambda i: (i, 0)),
            pl.BlockSpec((E, H), lambda i: (0, 0)),
            pl.BlockSpec((1, H), lambda i: (0, 0)),
            pl.BlockSpec((H, H), lambda i: (0, 0)),
            pl.BlockSpec((1, H), lambda i: (0, 0)),
            pl.BlockSpec((1, H), lambda i: (0, 0)),
            pl.BlockSpec((1, 1), lambda i: (0, 0)),
            pl.BlockSpec((H, n_classes), lambda i: (0, 0)),
            pl.BlockSpec((1, n_classes), lambda i: (0, 0)),
        ],
        out_specs=[
            pl.BlockSpec((bn, 1), lambda i: (i, 0)),
            pl.BlockSpec((1, n_classes), lambda i: (0, 0)),
            pl.BlockSpec((1, n_classes), lambda i: (0, 0)),
            pl.BlockSpec((1, 1), lambda i: (0, 0)),
        ],
        out_shape=[
            jax.ShapeDtypeStruct((N, 1), jnp.float32),
            jax.ShapeDtypeStruct((1, n_classes), jnp.float32),
            jax.ShapeDtypeStruct((1, n_classes), jnp.float32),
            jax.ShapeDtypeStruct((1, 1), jnp.int32),
        ],
        scratch_shapes=[
            pltpu.SMEM((1, 1), jnp.float32),
            pltpu.VMEM((1, H), jnp.float32),
        ],
        compiler_params=pltpu.CompilerParams(
            dimension_semantics=("arbitrary",),
        ),
    )(h, w1t, b1r, wbt, bbr, wcr, bcr, wclst, bclsr)

    return logits, yprob, yhat, araw.reshape(1, N)


# speedup vs baseline: 1.7232x; 1.0584x over previous
import functools

import jax
import jax.numpy as jnp
from jax.experimental import pallas as pl
from jax.experimental.pallas import tpu as pltpu


def _mil_body(nb, bn, h_ref, w1_ref, b1_ref, wb_ref, bb_ref, wc_ref,
              bc_ref, wcls_ref, bcls_ref,
              araw_ref, logits_ref, yprob_ref, yhat_ref,
              bestv_ref, bestf_ref, w1t_ref, wbt_ref):
    i = pl.program_id(0)

    @pl.when(i == 0)
    def _init():
        bestv_ref[0, 0] = -jnp.inf
        w1t_ref[...] = w1_ref[...].astype(jnp.bfloat16).T
        wbt_ref[...] = wb_ref[...].astype(jnp.bfloat16).T

    x = jnp.dot(h_ref[...].astype(jnp.bfloat16), w1t_ref[...],
                preferred_element_type=jnp.float32)
    x = jnp.maximum(x + b1_ref[...], 0.0)
    gate = jnp.dot(x.astype(jnp.bfloat16), wbt_ref[...],
                   preferred_element_type=jnp.float32)
    gate = jax.nn.sigmoid(gate + bb_ref[...])
    feat = x * gate
    a = jnp.sum(feat * wc_ref[...], axis=1, keepdims=True)
    a = a + bc_ref[0, 0]
    araw_ref[...] = a

    av = a[:, 0]
    bmax = jnp.max(av)
    bidx = jnp.argmax(av)

    @pl.when(bmax > bestv_ref[0, 0])
    def _update():
        bestv_ref[0, 0] = bmax
        rows = jax.lax.broadcasted_iota(jnp.int32, feat.shape, 0)
        bestf_ref[...] = jnp.sum(
            jnp.where(rows == bidx, feat, 0.0), axis=0, keepdims=True)

    @pl.when(i == nb - 1)
    def _finish():
        m = bestf_ref[...]
        logits = jax.lax.dot_general(
            m, wcls_ref[...], (((1,), (1,)), ((), ())),
            preferred_element_type=jnp.float32) + bcls_ref[...]
        logits_ref[...] = logits
        yprob_ref[...] = jax.nn.softmax(logits, axis=1)
        yhat_ref[...] = jnp.argmax(logits, axis=1).reshape(1, 1).astype(jnp.int32)


@jax.jit
def kernel(h, W1, b1, Wb, bb, Wc, bc, Wcls, bcls):
    N, E = h.shape
    H = W1.shape[0]
    n_classes = Wcls.shape[0]

    bn = 5000 if N % 5000 == 0 else None
    if bn is None:
        for cand in (2000, 1000, 500, 400, 250, 200, 125, 100, 50, 25, 8, 1):
            if N % cand == 0:
                bn = cand
                break
    nb = N // bn

    b1r = b1.reshape(1, H)
    bbr = bb.reshape(1, H)
    wcr = Wc.reshape(1, H)
    bcr = bc.reshape(1, 1)
    bclsr = bcls.reshape(1, n_classes)

    araw, logits, yprob, yhat = pl.pallas_call(
        functools.partial(_mil_body, nb, bn),
        grid=(nb,),
        in_specs=[
            pl.BlockSpec((bn, E), lambda i: (i, 0)),
            pl.BlockSpec((H, E), lambda i: (0, 0)),
            pl.BlockSpec((1, H), lambda i: (0, 0)),
            pl.BlockSpec((H, H), lambda i: (0, 0)),
            pl.BlockSpec((1, H), lambda i: (0, 0)),
            pl.BlockSpec((1, H), lambda i: (0, 0)),
            pl.BlockSpec((1, 1), lambda i: (0, 0)),
            pl.BlockSpec((n_classes, H), lambda i: (0, 0)),
            pl.BlockSpec((1, n_classes), lambda i: (0, 0)),
        ],
        out_specs=[
            pl.BlockSpec((bn, 1), lambda i: (i, 0)),
            pl.BlockSpec((1, n_classes), lambda i: (0, 0)),
            pl.BlockSpec((1, n_classes), lambda i: (0, 0)),
            pl.BlockSpec((1, 1), lambda i: (0, 0)),
        ],
        out_shape=[
            jax.ShapeDtypeStruct((N, 1), jnp.float32),
            jax.ShapeDtypeStruct((1, n_classes), jnp.float32),
            jax.ShapeDtypeStruct((1, n_classes), jnp.float32),
            jax.ShapeDtypeStruct((1, 1), jnp.int32),
        ],
        scratch_shapes=[
            pltpu.SMEM((1, 1), jnp.float32),
            pltpu.VMEM((1, H), jnp.float32),
            pltpu.VMEM((E, H), jnp.bfloat16),
            pltpu.VMEM((H, H), jnp.bfloat16),
        ],
        compiler_params=pltpu.CompilerParams(
            dimension_semantics=("arbitrary",),
        ),
    )(h, W1, b1r, Wb, bbr, wcr, bcr, Wcls, bclsr)

    return logits, yprob, yhat, araw.reshape(1, N)


# speedup vs baseline: 1.9741x; 1.1456x over previous
import functools

import jax
import jax.numpy as jnp
from jax.experimental import pallas as pl
from jax.experimental.pallas import tpu as pltpu


def _mil_body(nb, bn, h_ref, w1_ref, b1_ref, wb_ref, bb_ref, wc_ref,
              bc_ref, wcls_ref, bcls_ref,
              araw_ref, logits_ref, yprob_ref, yhat_ref,
              bestv_ref, bestf_ref, w1t_ref, wbt_ref):
    i = pl.program_id(0)

    @pl.when(i == 0)
    def _init():
        bestv_ref[0, 0] = -jnp.inf
        w1t_ref[...] = w1_ref[...].astype(jnp.bfloat16).T
        wbt_ref[...] = wb_ref[...].astype(jnp.bfloat16).T

    x = jnp.dot(h_ref[...].astype(jnp.bfloat16), w1t_ref[...],
                preferred_element_type=jnp.float32)
    x = jnp.maximum(x + b1_ref[...], 0.0)
    gate = jnp.dot(x.astype(jnp.bfloat16), wbt_ref[...],
                   preferred_element_type=jnp.float32)
    gate = jax.nn.sigmoid(gate + bb_ref[...])
    feat = x * gate
    a = jnp.sum(feat * wc_ref[...], axis=1, keepdims=True)
    a = a + bc_ref[0, 0]
    araw_ref[...] = a.T.reshape(1, 1, -1)

    av = a[:, 0]
    bmax = jnp.max(av)
    bidx = jnp.argmax(av)

    @pl.when(bmax > bestv_ref[0, 0])
    def _update():
        bestv_ref[0, 0] = bmax
        rows = jax.lax.broadcasted_iota(jnp.int32, feat.shape, 0)
        bestf_ref[...] = jnp.sum(
            jnp.where(rows == bidx, feat, 0.0), axis=0, keepdims=True)

    @pl.when(i == nb - 1)
    def _finish():
        m = bestf_ref[...]
        logits = jax.lax.dot_general(
            m, wcls_ref[...], (((1,), (1,)), ((), ())),
            preferred_element_type=jnp.float32) + bcls_ref[...]
        logits_ref[...] = logits
        yprob_ref[...] = jax.nn.softmax(logits, axis=1)
        yhat_ref[...] = jnp.argmax(logits, axis=1).reshape(1, 1).astype(jnp.int32)


@jax.jit
def kernel(h, W1, b1, Wb, bb, Wc, bc, Wcls, bcls):
    N, E = h.shape
    H = W1.shape[0]
    n_classes = Wcls.shape[0]

    bn = 5000 if N % 5000 == 0 else None
    if bn is None:
        for cand in (2000, 1000, 500, 400, 250, 200, 125, 100, 50, 25, 8, 1):
            if N % cand == 0:
                bn = cand
                break
    nb = N // bn

    b1r = b1.reshape(1, H)
    bbr = bb.reshape(1, H)
    wcr = Wc.reshape(1, H)
    bcr = bc.reshape(1, 1)
    bclsr = bcls.reshape(1, n_classes)

    araw, logits, yprob, yhat = pl.pallas_call(
        functools.partial(_mil_body, nb, bn),
        grid=(nb,),
        in_specs=[
            pl.BlockSpec((bn, E), lambda i: (i, 0)),
            pl.BlockSpec((H, E), lambda i: (0, 0)),
            pl.BlockSpec((1, H), lambda i: (0, 0)),
            pl.BlockSpec((H, H), lambda i: (0, 0)),
            pl.BlockSpec((1, H), lambda i: (0, 0)),
            pl.BlockSpec((1, H), lambda i: (0, 0)),
            pl.BlockSpec((1, 1), lambda i: (0, 0)),
            pl.BlockSpec((n_classes, H), lambda i: (0, 0)),
            pl.BlockSpec((1, n_classes), lambda i: (0, 0)),
        ],
        out_specs=[
            pl.BlockSpec((1, 1, bn), lambda i: (i, 0, 0)),
            pl.BlockSpec((1, n_classes), lambda i: (0, 0)),
            pl.BlockSpec((1, n_classes), lambda i: (0, 0)),
            pl.BlockSpec((1, 1), lambda i: (0, 0)),
        ],
        out_shape=[
            jax.ShapeDtypeStruct((nb, 1, bn), jnp.float32),
            jax.ShapeDtypeStruct((1, n_classes), jnp.float32),
            jax.ShapeDtypeStruct((1, n_classes), jnp.float32),
            jax.ShapeDtypeStruct((1, 1), jnp.int32),
        ],
        scratch_shapes=[
            pltpu.SMEM((1, 1), jnp.float32),
            pltpu.VMEM((1, H), jnp.float32),
            pltpu.VMEM((E, H), jnp.bfloat16),
            pltpu.VMEM((H, H), jnp.bfloat16),
        ],
        compiler_params=pltpu.CompilerParams(
            dimension_semantics=("arbitrary",),
        ),
    )(h, W1, b1r, Wb, bbr, wcr, bcr, Wcls, bclsr)

    return logits, yprob, yhat, araw.reshape(1, N)


# speedup vs baseline: 2.0550x; 1.0410x over previous
import functools

import jax
import jax.numpy as jnp
from jax.experimental import pallas as pl
from jax.experimental.pallas import tpu as pltpu


def _mil_body(nb, bn, h_ref, w1_ref, b1_ref, wb_ref, bb_ref, wc_ref,
              bc_ref, wcls_ref, bcls_ref,
              araw_ref, logits_ref, yprob_ref, yhat_ref,
              bestv_ref, bestf_ref, w1t_ref, wbt_ref, arows_ref):
    i = pl.program_id(0)

    @pl.when(i == 0)
    def _init():
        bestv_ref[0, 0] = -jnp.inf
        w1t_ref[...] = w1_ref[...].astype(jnp.bfloat16).T
        wbt_ref[...] = wb_ref[...].astype(jnp.bfloat16).T

    x = jnp.dot(h_ref[...].astype(jnp.bfloat16), w1t_ref[...],
                preferred_element_type=jnp.float32)
    x = jnp.maximum(x + b1_ref[...], 0.0)
    gate = jnp.dot(x.astype(jnp.bfloat16), wbt_ref[...],
                   preferred_element_type=jnp.float32)
    gate = jax.nn.sigmoid(gate + bb_ref[...])
    feat = x * gate
    a = jnp.sum(feat * wc_ref[...], axis=1, keepdims=True)
    a = a + bc_ref[0, 0]
    arows_ref[i, :, :] = a.T

    av = a[:, 0]
    bmax = jnp.max(av)
    bidx = jnp.argmax(av)

    @pl.when(bmax > bestv_ref[0, 0])
    def _update():
        bestv_ref[0, 0] = bmax
        rows = jax.lax.broadcasted_iota(jnp.int32, feat.shape, 0)
        bestf_ref[...] = jnp.sum(
            jnp.where(rows == bidx, feat, 0.0), axis=0, keepdims=True)

    @pl.when(i == nb - 1)
    def _finish():
        m = bestf_ref[...]
        logits = jax.lax.dot_general(
            m, wcls_ref[...], (((1,), (1,)), ((), ())),
            preferred_element_type=jnp.float32) + bcls_ref[...]
        logits_ref[...] = logits
        yprob_ref[...] = jax.nn.softmax(logits, axis=1)
        yhat_ref[...] = jnp.argmax(logits, axis=1).reshape(1, 1).astype(jnp.int32)
        araw_ref[...] = jnp.concatenate(
            [arows_ref[k, :, :] for k in range(nb)], axis=1)


@jax.jit
def kernel(h, W1, b1, Wb, bb, Wc, bc, Wcls, bcls):
    N, E = h.shape
    H = W1.shape[0]
    n_classes = Wcls.shape[0]

    bn = 5000 if N % 5000 == 0 else None
    if bn is None:
        for cand in (2000, 1000, 500, 400, 250, 200, 125, 100, 50, 25, 8, 1):
            if N % cand == 0:
                bn = cand
                break
    nb = N // bn

    b1r = b1.reshape(1, H)
    bbr = bb.reshape(1, H)
    wcr = Wc.reshape(1, H)
    bcr = bc.reshape(1, 1)
    bclsr = bcls.reshape(1, n_classes)

    araw, logits, yprob, yhat = pl.pallas_call(
        functools.partial(_mil_body, nb, bn),
        grid=(nb,),
        in_specs=[
            pl.BlockSpec((bn, E), lambda i: (i, 0)),
            pl.BlockSpec((H, E), lambda i: (0, 0)),
            pl.BlockSpec((1, H), lambda i: (0, 0)),
            pl.BlockSpec((H, H), lambda i: (0, 0)),
            pl.BlockSpec((1, H), lambda i: (0, 0)),
            pl.BlockSpec((1, H), lambda i: (0, 0)),
            pl.BlockSpec((1, 1), lambda i: (0, 0)),
            pl.BlockSpec((n_classes, H), lambda i: (0, 0)),
            pl.BlockSpec((1, n_classes), lambda i: (0, 0)),
        ],
        out_specs=[
            pl.BlockSpec((1, N), lambda i: (0, 0)),
            pl.BlockSpec((1, n_classes), lambda i: (0, 0)),
            pl.BlockSpec((1, n_classes), lambda i: (0, 0)),
            pl.BlockSpec((1, 1), lambda i: (0, 0)),
        ],
        out_shape=[
            jax.ShapeDtypeStruct((1, N), jnp.float32),
            jax.ShapeDtypeStruct((1, n_classes), jnp.float32),
            jax.ShapeDtypeStruct((1, n_classes), jnp.float32),
            jax.ShapeDtypeStruct((1, 1), jnp.int32),
        ],
        scratch_shapes=[
            pltpu.SMEM((1, 1), jnp.float32),
            pltpu.VMEM((1, H), jnp.float32),
            pltpu.VMEM((E, H), jnp.bfloat16),
            pltpu.VMEM((H, H), jnp.bfloat16),
            pltpu.VMEM((nb, 1, bn), jnp.float32),
        ],
        compiler_params=pltpu.CompilerParams(
            dimension_semantics=("arbitrary",),
        ),
    )(h, W1, b1r, Wb, bbr, wcr, bcr, Wcls, bclsr)

    return logits, yprob, yhat, araw
